# baseline (speedup 1.0000x reference)
import jax
import jax.numpy as jnp
from jax.experimental import pallas as pl
from jax.experimental.pallas import tpu as pltpu

_DIM = 512
_N_BUCKETS = 1024
_BANDWIDTH = 4.0
_N_HASHES = 128


def _lsh_block_kernel(x_ref, rv_ref, out_ref):
    proj = jnp.dot(x_ref[...], rv_ref[...], preferred_element_type=jnp.float32)
    buckets = jnp.floor(proj * (1.0 / _BANDWIDTH)).astype(jnp.int32)
    out_ref[...] = (buckets & (_N_BUCKETS - 1)).astype(jnp.float32)


def kernel(x, random_vectors):
    n = x.shape[0]
    block = 7168
    return pl.pallas_call(
        _lsh_block_kernel,
        grid=(n // block,),
        in_specs=[
            pl.BlockSpec((block, _DIM), lambda i: (i, 0)),
            pl.BlockSpec((_DIM, _N_HASHES), lambda i: (0, 0)),
        ],
        out_specs=pl.BlockSpec((block, _N_HASHES), lambda i: (i, 0)),
        out_shape=jax.ShapeDtypeStruct((n, _N_HASHES), jnp.float32),
        compiler_params=pltpu.CompilerParams(
            dimension_semantics=("parallel",),
        ),
    )(x, random_vectors)


# speedup vs baseline: 1.0052x; 1.0052x over previous
import jax
import jax.numpy as jnp
from jax.experimental import pallas as pl
from jax.experimental.pallas import tpu as pltpu

_DIM = 512
_N_BUCKETS = 1024
_BANDWIDTH = 4.0
_N_HASHES = 128


def _lsh_block_kernel(x_ref, rv_ref, out_ref):
    proj = jnp.dot(x_ref[...], rv_ref[...], preferred_element_type=jnp.float32)
    buckets = jnp.floor(proj * (1.0 / _BANDWIDTH)).astype(jnp.int32)
    out_ref[...] = (buckets & (_N_BUCKETS - 1)).astype(jnp.float32)


def kernel(x, random_vectors):
    n = x.shape[0]
    block = 5376
    return pl.pallas_call(
        _lsh_block_kernel,
        grid=(n // block,),
        in_specs=[
            pl.BlockSpec((block, _DIM), lambda i: (i, 0)),
            pl.BlockSpec((_DIM, _N_HASHES), lambda i: (0, 0)),
        ],
        out_specs=pl.BlockSpec((block, _N_HASHES), lambda i: (i, 0)),
        out_shape=jax.ShapeDtypeStruct((n, _N_HASHES), jnp.float32),
        compiler_params=pltpu.CompilerParams(
            dimension_semantics=("parallel",),
        ),
    )(x, random_vectors)


# speedup vs baseline: 1.0747x; 1.0691x over previous
import jax
import jax.numpy as jnp
from jax.experimental import pallas as pl
from jax.experimental.pallas import tpu as pltpu

_DIM = 512
_N_BUCKETS = 1024
_BANDWIDTH = 4.0
_N_HASHES = 128


def _lsh_block_kernel(x_ref, rv_ref, out_ref):
    proj = jnp.dot(x_ref[...], rv_ref[...], preferred_element_type=jnp.float32)
    buckets = jnp.floor(proj * (1.0 / _BANDWIDTH)).astype(jnp.int32)
    out_ref[...] = (buckets & (_N_BUCKETS - 1)).astype(jnp.float32)


def kernel(x, random_vectors):
    n = x.shape[0]
    block = 5464
    return pl.pallas_call(
        _lsh_block_kernel,
        grid=(n // block,),
        in_specs=[
            pl.BlockSpec((block, _DIM), lambda i: (i, 0)),
            pl.BlockSpec((_DIM, _N_HASHES), lambda i: (0, 0)),
        ],
        out_specs=pl.BlockSpec((block, _N_HASHES), lambda i: (i, 0)),
        out_shape=jax.ShapeDtypeStruct((n, _N_HASHES), jnp.float32),
        compiler_params=pltpu.CompilerParams(
            dimension_semantics=("parallel",),
        ),
    )(x, random_vectors)


# speedup vs baseline: 1.0827x; 1.0075x over previous
import jax
import jax.numpy as jnp
from jax.experimental import pallas as pl
from jax.experimental.pallas import tpu as pltpu

_DIM = 512
_N_BUCKETS = 1024
_BANDWIDTH = 4.0
_N_HASHES = 128


def _lsh_block_kernel(x_ref, rv_ref, out_ref):
    proj = jnp.dot(x_ref[...], rv_ref[...], preferred_element_type=jnp.float32)
    buckets = jnp.floor(proj * (1.0 / _BANDWIDTH)).astype(jnp.int32)
    out_ref[...] = (buckets & (_N_BUCKETS - 1)).astype(jnp.float32)


def kernel(x, random_vectors):
    n = x.shape[0]
    block = 5960
    return pl.pallas_call(
        _lsh_block_kernel,
        grid=(n // block,),
        in_specs=[
            pl.BlockSpec((block, _DIM), lambda i: (i, 0)),
            pl.BlockSpec((_DIM, _N_HASHES), lambda i: (0, 0)),
        ],
        out_specs=pl.BlockSpec((block, _N_HASHES), lambda i: (i, 0)),
        out_shape=jax.ShapeDtypeStruct((n, _N_HASHES), jnp.float32),
        compiler_params=pltpu.CompilerParams(
            dimension_semantics=("parallel",),
        ),
    )(x, random_vectors)


# speedup vs baseline: 1.0909x; 1.0076x over previous
import jax
import jax.numpy as jnp
from jax.experimental import pallas as pl
from jax.experimental.pallas import tpu as pltpu

_DIM = 512
_N_BUCKETS = 1024
_BANDWIDTH = 4.0
_N_HASHES = 128


def _lsh_block_kernel(x_ref, rv_ref, out_ref):
    proj = jnp.dot(x_ref[...], rv_ref[...], preferred_element_type=jnp.float32)
    buckets = jnp.floor(proj * (1.0 / _BANDWIDTH)).astype(jnp.int32)
    out_ref[...] = (buckets & (_N_BUCKETS - 1)).astype(jnp.float32)


def kernel(x, random_vectors):
    n = x.shape[0]
    block = 6560
    return pl.pallas_call(
        _lsh_block_kernel,
        grid=(n // block,),
        in_specs=[
            pl.BlockSpec((block, _DIM), lambda i: (i, 0)),
            pl.BlockSpec((_DIM, _N_HASHES), lambda i: (0, 0)),
        ],
        out_specs=pl.BlockSpec((block, _N_HASHES), lambda i: (i, 0)),
        out_shape=jax.ShapeDtypeStruct((n, _N_HASHES), jnp.float32),
        compiler_params=pltpu.CompilerParams(
            dimension_semantics=("parallel",),
        ),
    )(x, random_vectors)


# speedup vs baseline: 1.0941x; 1.0029x over previous
import jax
import jax.numpy as jnp
from jax.experimental import pallas as pl
from jax.experimental.pallas import tpu as pltpu

_DIM = 512
_N_BUCKETS = 1024
_BANDWIDTH = 4.0
_N_HASHES = 128


def _lsh_block_kernel(x_ref, rv_ref, out_ref):
    proj = jnp.dot(x_ref[...], rv_ref[...], preferred_element_type=jnp.float32)
    buckets = jnp.floor(proj * (1.0 / _BANDWIDTH)).astype(jnp.int32)
    out_ref[...] = (buckets & (_N_BUCKETS - 1)).astype(jnp.float32)


def kernel(x, random_vectors):
    n = x.shape[0]
    block = 7288
    return pl.pallas_call(
        _lsh_block_kernel,
        grid=(n // block,),
        in_specs=[
            pl.BlockSpec((block, _DIM), lambda i: (i, 0)),
            pl.BlockSpec((_DIM, _N_HASHES), lambda i: (0, 0)),
        ],
        out_specs=pl.BlockSpec((block, _N_HASHES), lambda i: (i, 0)),
        out_shape=jax.ShapeDtypeStruct((n, _N_HASHES), jnp.float32),
        compiler_params=pltpu.CompilerParams(
            dimension_semantics=("parallel",),
        ),
    )(x, random_vectors)


# speedup vs baseline: 1.1119x; 1.0162x over previous
import jax
import jax.numpy as jnp
from jax.experimental import pallas as pl
from jax.experimental.pallas import tpu as pltpu

_DIM = 512
_N_BUCKETS = 1024
_BANDWIDTH = 4.0
_N_HASHES = 128


def _lsh_block_kernel(x_ref, rv_ref, out_ref):
    proj = jnp.dot(x_ref[...], rv_ref[...], preferred_element_type=jnp.float32)
    buckets = jnp.floor(proj * (1.0 / _BANDWIDTH)).astype(jnp.int32)
    out_ref[...] = (buckets & (_N_BUCKETS - 1)).astype(jnp.float32)


def kernel(x, random_vectors):
    n = x.shape[0]
    block = 8200
    return pl.pallas_call(
        _lsh_block_kernel,
        grid=(n // block,),
        in_specs=[
            pl.BlockSpec((block, _DIM), lambda i: (i, 0)),
            pl.BlockSpec((_DIM, _N_HASHES), lambda i: (0, 0)),
        ],
        out_specs=pl.BlockSpec((block, _N_HASHES), lambda i: (i, 0)),
        out_shape=jax.ShapeDtypeStruct((n, _N_HASHES), jnp.float32),
        compiler_params=pltpu.CompilerParams(
            dimension_semantics=("parallel",),
        ),
    )(x, random_vectors)


# speedup vs baseline: 1.1181x; 1.0056x over previous
import jax
import jax.numpy as jnp
from jax.experimental import pallas as pl
from jax.experimental.pallas import tpu as pltpu

_DIM = 512
_N_BUCKETS = 1024
_BANDWIDTH = 4.0
_N_HASHES = 128


def _lsh_block_kernel(x_ref, rv_ref, out_ref):
    proj = jnp.dot(x_ref[...], rv_ref[...], preferred_element_type=jnp.float32)
    buckets = jnp.floor(proj * (1.0 / _BANDWIDTH)).astype(jnp.int32)
    out_ref[...] = (buckets & (_N_BUCKETS - 1)).astype(jnp.float32)


def kernel(x, random_vectors):
    n = x.shape[0]
    block = 9368
    return pl.pallas_call(
        _lsh_block_kernel,
        grid=(n // block,),
        in_specs=[
            pl.BlockSpec((block, _DIM), lambda i: (i, 0)),
            pl.BlockSpec((_DIM, _N_HASHES), lambda i: (0, 0)),
        ],
        out_specs=pl.BlockSpec((block, _N_HASHES), lambda i: (i, 0)),
        out_shape=jax.ShapeDtypeStruct((n, _N_HASHES), jnp.float32),
        compiler_params=pltpu.CompilerParams(
            dimension_semantics=("parallel",),
        ),
    )(x, random_vectors)


# speedup vs baseline: 1.1367x; 1.0166x over previous
import jax
import jax.numpy as jnp
from jax.experimental import pallas as pl
from jax.experimental.pallas import tpu as pltpu

_DIM = 512
_N_BUCKETS = 1024
_BANDWIDTH = 4.0
_N_HASHES = 128


def _lsh_block_kernel(x_ref, rv_ref, out_ref):
    proj = jnp.dot(x_ref[...], rv_ref[...], preferred_element_type=jnp.float32)
    buckets = jnp.floor(proj * (1.0 / _BANDWIDTH)).astype(jnp.int32)
    out_ref[...] = (buckets & (_N_BUCKETS - 1)).astype(jnp.float32)


def kernel(x, random_vectors):
    n = x.shape[0]
    block = 10928
    return pl.pallas_call(
        _lsh_block_kernel,
        grid=(n // block,),
        in_specs=[
            pl.BlockSpec((block, _DIM), lambda i: (i, 0)),
            pl.BlockSpec((_DIM, _N_HASHES), lambda i: (0, 0)),
        ],
        out_specs=pl.BlockSpec((block, _N_HASHES), lambda i: (i, 0)),
        out_shape=jax.ShapeDtypeStruct((n, _N_HASHES), jnp.float32),
        compiler_params=pltpu.CompilerParams(
            dimension_semantics=("parallel",),
        ),
    )(x, random_vectors)
